# baseline (speedup 1.0000x reference)
import functools

import jax
import jax.numpy as jnp
from jax import lax
from jax.experimental import pallas as pl
from jax.experimental.pallas import tpu as pltpu
from jax.experimental.pallas import tpu_sc as plsc

_B, _S, _D = 16384, 50, 128
_SP = 56
_NC, _NS = 2, 16
_NW = _NC * _NS
_BPC = 2
_CH = _BPC * _SP
_PER_W = _B // _NW
_NCH = _PER_W // _BPC
_NBUF = 6
_W = _NBUF // 2
_TCB = 64


def _emb_body(ids_hbm, table_hbm, out_hbm, idx_v, *rest):
    bufs = rest[:_NBUF]
    gsems = rest[_NBUF:2 * _NBUF]
    wsems = rest[2 * _NBUF:]
    wid = lax.axis_index("s") * _NC + lax.axis_index("c")
    row0 = wid * _PER_W * _SP

    pltpu.sync_copy(ids_hbm.at[wid], idx_v)

    def out_at(j):
        return out_hbm.at[pl.ds(row0 + j * _CH, _CH)]

    def start_gather(j, k):
        pltpu.async_copy(table_hbm.at[idx_v.at[j]], bufs[k], gsems[k])

    def wait_gather(j, k):
        pltpu.make_async_copy(table_hbm.at[idx_v.at[j]], bufs[k], gsems[k]).wait()

    def start_write(j, k):
        pltpu.async_copy(bufs[k], out_at(j), wsems[k])

    def wait_write(j, k):
        pltpu.make_async_copy(bufs[k], out_at(j), wsems[k]).wait()

    def step(j, k, prefetch, wait_w):
        wait_gather(j, k)
        start_write(j, k)
        if prefetch:
            k2 = (k + _W) % _NBUF
            if wait_w:
                wait_write(j + _W - _NBUF, k2)
            start_gather(j + _W, k2)

    for j in range(_W):
        start_gather(j, j % _NBUF)

    head_end = _NBUF - _W
    for j in range(head_end):
        step(j, j % _NBUF, prefetch=True, wait_w=False)

    n_steady = _NCH - _W - head_end
    n_groups = n_steady // _NBUF

    def body(i, carry):
        j0 = _NBUF * i + head_end
        for r in range(_NBUF):
            step(j0 + r, (head_end + r) % _NBUF, prefetch=True, wait_w=True)
        return carry

    lax.fori_loop(0, n_groups, body, 0)

    for j in range(head_end + n_groups * _NBUF, _NCH - _W):
        step(j, j % _NBUF, prefetch=True, wait_w=True)

    for j in range(_NCH - _W, _NCH):
        step(j, j % _NBUF, prefetch=False, wait_w=False)

    for j in range(_NCH - _NBUF, _NCH):
        wait_write(j, j % _NBUF)


def _sc_gather(ids, weight):
    mesh = plsc.VectorSubcoreMesh(core_axis_name="c", subcore_axis_name="s")
    return pl.kernel(
        _emb_body,
        mesh=mesh,
        out_type=jax.ShapeDtypeStruct((_B * _SP, _D), jnp.float32),
        scratch_types=(
            [pltpu.VMEM((_NCH, _CH), jnp.int32)]
            + [pltpu.VMEM((_CH, _D), jnp.float32)] * _NBUF
            + [pltpu.SemaphoreType.DMA] * (2 * _NBUF)
        ),
    )(ids, weight)


def _compact_body(pad_ref, out_ref):
    out_ref[...] = pad_ref[:, : _S, :]


def _tc_compact(padded):
    return pl.pallas_call(
        _compact_body,
        grid=(_B // _TCB,),
        in_specs=[pl.BlockSpec((_TCB, _SP, _D), lambda i: (i, 0, 0))],
        out_specs=pl.BlockSpec((_TCB, _S, _D), lambda i: (i, 0, 0)),
        out_shape=jax.ShapeDtypeStruct((_B, _S, _D), jnp.float32),
    )(padded)


@jax.jit
def kernel(token_ids, weight):
    ids = token_ids.astype(jnp.int32)
    ids = jnp.pad(ids, ((0, 0), (0, _SP - _S)), mode="edge")
    ids = ids.reshape(_NW, _NCH, _CH)
    padded = _sc_gather(ids, weight)
    return _tc_compact(padded.reshape(_B, _SP, _D))


# speedup vs baseline: 1.6354x; 1.6354x over previous
import jax
import jax.numpy as jnp
from jax import lax
from jax.experimental import pallas as pl
from jax.experimental.pallas import tpu as pltpu
from jax.experimental.pallas import tpu_sc as plsc

_B, _S, _D = 16384, 50, 128
_SP = 56
_NC, _NS = 2, 16
_NW = _NC * _NS
_BPC = 2
_CH = _BPC * _SP
_PER_W = _B // _NW
_NCH = _PER_W // _BPC
_NBUF = 4
_W = _NBUF // 2


def _emb_body(ids_hbm, table_hbm, out_hbm, idx_v, *rest):
    bufs = rest[:_NBUF]
    gsems = rest[_NBUF:2 * _NBUF]
    wsems = rest[2 * _NBUF:]
    wid = lax.axis_index("s") * _NC + lax.axis_index("c")
    bat0 = wid * _PER_W

    pltpu.sync_copy(ids_hbm.at[wid], idx_v)

    def start_gather(j, k):
        for t in range(_BPC):
            pltpu.async_copy(table_hbm.at[idx_v.at[j, t, pl.ds(0, _S)]],
                             bufs[k].at[t], gsems[k])

    def wait_gather(j, k):
        pltpu.make_async_copy(out_hbm.at[pl.ds(0, _BPC)], bufs[k],
                              gsems[k]).wait()

    def start_write(j, k):
        pltpu.async_copy(bufs[k],
                         out_hbm.at[pl.ds(bat0 + _BPC * j, _BPC)], wsems[k])

    def wait_write(j, k):
        pltpu.make_async_copy(bufs[k],
                              out_hbm.at[pl.ds(bat0 + _BPC * j, _BPC)],
                              wsems[k]).wait()

    def step(j, k, prefetch, wait_w):
        wait_gather(j, k)
        start_write(j, k)
        if prefetch:
            k2 = (k + _W) % _NBUF
            if wait_w:
                wait_write(j + _W - _NBUF, k2)
            start_gather(j + _W, k2)

    for j in range(_W):
        start_gather(j, j % _NBUF)

    head_end = _NBUF - _W
    for j in range(head_end):
        step(j, j % _NBUF, prefetch=True, wait_w=False)

    n_steady = _NCH - _W - head_end
    n_groups = n_steady // _NBUF

    def body(i, carry):
        j0 = _NBUF * i + head_end
        for r in range(_NBUF):
            step(j0 + r, (head_end + r) % _NBUF, prefetch=True, wait_w=True)
        return carry

    lax.fori_loop(0, n_groups, body, 0)

    for j in range(head_end + n_groups * _NBUF, _NCH - _W):
        step(j, j % _NBUF, prefetch=True, wait_w=True)

    for j in range(_NCH - _W, _NCH):
        step(j, j % _NBUF, prefetch=False, wait_w=False)

    for j in range(_NCH - _NBUF, _NCH):
        wait_write(j, j % _NBUF)


@jax.jit
def kernel(token_ids, weight):
    ids = token_ids.astype(jnp.int32)
    ids = jnp.pad(ids, ((0, 0), (0, _SP - _S)), mode="edge")
    ids = ids.reshape(_NW, _NCH, _BPC, _SP)
    mesh = plsc.VectorSubcoreMesh(core_axis_name="c", subcore_axis_name="s")
    out = pl.kernel(
        _emb_body,
        mesh=mesh,
        out_type=jax.ShapeDtypeStruct((_B, _S, _D), jnp.float32),
        scratch_types=(
            [pltpu.VMEM((_NCH, _BPC, _SP), jnp.int32)]
            + [pltpu.VMEM((_BPC, _S, _D), jnp.float32)] * _NBUF
            + [pltpu.SemaphoreType.DMA] * (2 * _NBUF)
        ),
    )(ids, weight)
    return out


# speedup vs baseline: 1.6362x; 1.0005x over previous
import jax
import jax.numpy as jnp
from jax import lax
from jax.experimental import pallas as pl
from jax.experimental.pallas import tpu as pltpu
from jax.experimental.pallas import tpu_sc as plsc

_B, _S, _D = 16384, 50, 128
_SP = 56
_NC, _NS = 2, 16
_NW = _NC * _NS
_BPC = 2
_PER_W = _B // _NW
_NCH = _PER_W // _BPC
_NBUF = 4
_W = _NBUF // 2


def _emb_body(ids_hbm, table_hbm, out_hbm, idx_v, *rest):
    bufs = rest[:_NBUF]
    gsems = rest[_NBUF:2 * _NBUF]
    wsems = rest[2 * _NBUF:]
    wid = lax.axis_index("s") * _NC + lax.axis_index("c")
    bat0 = wid * _PER_W

    pltpu.sync_copy(ids_hbm.at[wid], idx_v)

    def start_gather(j, k):
        for t in range(_BPC):
            pltpu.async_copy(table_hbm.at[idx_v.at[j, t, pl.ds(0, _S)]],
                             bufs[k].at[t], gsems[k])

    def wait_gather(j, k):
        pltpu.make_async_copy(out_hbm.at[pl.ds(0, _BPC)], bufs[k],
                              gsems[k]).wait()

    def start_write(j, k):
        pltpu.async_copy(bufs[k],
                         out_hbm.at[pl.ds(bat0 + _BPC * j, _BPC)], wsems[k])

    def wait_write(j, k):
        pltpu.make_async_copy(bufs[k],
                              out_hbm.at[pl.ds(bat0 + _BPC * j, _BPC)],
                              wsems[k]).wait()

    def step(j, k, prefetch, wait_w):
        wait_gather(j, k)
        start_write(j, k)
        if prefetch:
            k2 = (k + _W) % _NBUF
            if wait_w:
                wait_write(j + _W - _NBUF, k2)
            start_gather(j + _W, k2)

    for j in range(_W):
        start_gather(j, j % _NBUF)

    head_end = _NBUF - _W
    for j in range(head_end):
        step(j, j % _NBUF, prefetch=True, wait_w=False)

    n_steady = _NCH - _W - head_end
    n_groups = n_steady // _NBUF

    def body(i, carry):
        j0 = _NBUF * i + head_end
        for r in range(_NBUF):
            step(j0 + r, (head_end + r) % _NBUF, prefetch=True, wait_w=True)
        return carry

    lax.fori_loop(0, n_groups, body, 0)

    for j in range(head_end + n_groups * _NBUF, _NCH - _W):
        step(j, j % _NBUF, prefetch=True, wait_w=True)

    for j in range(_NCH - _W, _NCH):
        step(j, j % _NBUF, prefetch=False, wait_w=False)

    for j in range(_NCH - _NBUF, _NCH):
        wait_write(j, j % _NBUF)


@jax.jit
def kernel(token_ids, weight):
    ids = token_ids.astype(jnp.int32)
    ids = jnp.pad(ids, ((0, 0), (0, _SP - _S)), mode="edge")
    ids = ids.reshape(_NW, _NCH, _BPC, _SP)
    mesh = plsc.VectorSubcoreMesh(core_axis_name="c", subcore_axis_name="s")
    out = pl.kernel(
        _emb_body,
        mesh=mesh,
        out_type=jax.ShapeDtypeStruct((_B, _S, _D), jnp.float32),
        scratch_types=(
            [pltpu.VMEM((_NCH, _BPC, _SP), jnp.int32)]
            + [pltpu.VMEM((_BPC, _S, _D), jnp.float32)] * _NBUF
            + [pltpu.SemaphoreType.DMA] * (2 * _NBUF)
        ),
    )(ids, weight)
    return out
